# baseline (speedup 1.0000x reference)
import functools

import jax
import jax.numpy as jnp
from jax import lax
from jax.experimental import pallas as pl
from jax.experimental.pallas import tpu as pltpu
from jax.experimental.pallas import tpu_sc as plsc

BATCH = 4096
SEQ = 200
EMBED_DIM = 64
NUM_CORES = 2
NUM_SUBCORES = 16
NUM_WORKERS = NUM_CORES * NUM_SUBCORES
SEQ_PER_WORKER = BATCH // NUM_WORKERS
LANES = 16
VREGS_PER_ROW = EMBED_DIM // LANES


def _tpe_kernel(idx_hbm, table_hbm, pos_hbm, out_hbm, idx_v, rows_v, pos_v, sem):
    wid = lax.axis_index("s") * NUM_CORES + lax.axis_index("c")
    pltpu.sync_copy(pos_hbm, pos_v)

    def per_seq(i, carry):
        seq = wid * SEQ_PER_WORKER + i
        pltpu.sync_copy(idx_hbm.at[seq], idx_v)
        cp0 = pltpu.async_copy(
            table_hbm.at[idx_v.at[pl.ds(0, 128)]], rows_v.at[pl.ds(0, 128)], sem)
        cp1 = pltpu.async_copy(
            table_hbm.at[idx_v.at[pl.ds(128, 72)]], rows_v.at[pl.ds(128, 72)], sem)
        cp0.wait()
        cp1.wait()

        def add_row(r, c):
            for j in range(VREGS_PER_ROW):
                sl = pl.ds(j * LANES, LANES)
                rows_v[r, sl] = rows_v[r, sl] + pos_v[r, sl]
            return c

        lax.fori_loop(0, SEQ, add_row, 0, unroll=2)
        pltpu.sync_copy(rows_v, out_hbm.at[seq])
        return carry

    lax.fori_loop(0, SEQ_PER_WORKER, per_seq, 0)


def kernel(inputs, token_table, position_table):
    mesh = plsc.VectorSubcoreMesh(core_axis_name="c", subcore_axis_name="s")
    run = functools.partial(
        pl.kernel,
        out_type=jax.ShapeDtypeStruct((BATCH, SEQ, EMBED_DIM), jnp.float32),
        mesh=mesh,
        scratch_types=[
            pltpu.VMEM((SEQ,), jnp.int32),
            pltpu.VMEM((SEQ, EMBED_DIM), jnp.float32),
            pltpu.VMEM((SEQ, EMBED_DIM), jnp.float32),
            pltpu.SemaphoreType.DMA,
        ],
        compiler_params=pltpu.CompilerParams(use_tc_tiling_on_sc=False),
    )(_tpe_kernel)
    return run(inputs.astype(jnp.int32), token_table, position_table)


# speedup vs baseline: 1.1781x; 1.1781x over previous
import functools

import jax
import jax.numpy as jnp
from jax import lax
from jax.experimental import pallas as pl
from jax.experimental.pallas import tpu as pltpu
from jax.experimental.pallas import tpu_sc as plsc

BATCH = 4096
SEQ = 200
EMBED_DIM = 64
NUM_CORES = 2
NUM_SUBCORES = 16
NUM_WORKERS = NUM_CORES * NUM_SUBCORES
CHUNKS_PER_WORKER = BATCH // NUM_WORKERS
LANES = 16
VREGS_PER_ROW = EMBED_DIM // LANES
NBUF = 4
IDX_SPLITS = ((0, 128), (128, 72))


def _gather_descs(table_hbm, idx_v, buf, sem, k):
    off = k * SEQ
    return [
        pltpu.make_async_copy(
            table_hbm.at[idx_v.at[pl.ds(off + o, n)]], buf.at[pl.ds(o, n)], sem)
        for o, n in IDX_SPLITS
    ]


def _wb_desc(buf, out_hbm, sem, row_base):
    return pltpu.make_async_copy(buf, out_hbm.at[pl.ds(row_base, SEQ)], sem)


def _add_pos(buf, pos_v):
    def row(r, c):
        for j in range(VREGS_PER_ROW):
            sl = pl.ds(j * LANES, LANES)
            buf[r, sl] = buf[r, sl] + pos_v[r, sl]
        return c

    lax.fori_loop(0, SEQ, row, 0, unroll=4)


def _tpe_kernel(idx_hbm, table_hbm, pos_hbm, out_hbm,
                idx_v, pos_v, b0, b1, b2, b3, g0, g1, g2, g3, w0, w1, w2, w3):
    bufs = [b0, b1, b2, b3]
    gsems = [g0, g1, g2, g3]
    wsems = [w0, w1, w2, w3]
    wid = lax.axis_index("s") * NUM_CORES + lax.axis_index("c")
    chunk_base = wid * CHUNKS_PER_WORKER
    n = CHUNKS_PER_WORKER

    pltpu.sync_copy(pos_hbm, pos_v)
    pltpu.sync_copy(idx_hbm.at[pl.ds(chunk_base * SEQ, n * SEQ)], idx_v)

    for b in range(2):
        for d in _gather_descs(table_hbm, idx_v, bufs[b], gsems[b], b):
            d.start()

    def group(p, carry):
        for b in range(NBUF):
            k = NBUF * p + b
            nb = (b + 2) % NBUF

            @pl.when(jnp.logical_and(k >= 2, k + 2 < n))
            def _():
                _wb_desc(bufs[nb], out_hbm, wsems[nb],
                         (chunk_base + k - 2) * SEQ).wait()

            @pl.when(k + 2 < n)
            def _():
                for d in _gather_descs(table_hbm, idx_v, bufs[nb], gsems[nb],
                                       k + 2):
                    d.start()

            for d in _gather_descs(table_hbm, idx_v, bufs[b], gsems[b], k):
                d.wait()
            _add_pos(bufs[b], pos_v)
            _wb_desc(bufs[b], out_hbm, wsems[b], (chunk_base + k) * SEQ).start()
        return carry

    lax.fori_loop(0, n // NBUF, group, 0)

    for b in range(NBUF):
        _wb_desc(bufs[b], out_hbm, wsems[b],
                 (chunk_base + n - NBUF + b) * SEQ).wait()


def kernel(inputs, token_table, position_table):
    mesh = plsc.VectorSubcoreMesh(core_axis_name="c", subcore_axis_name="s")
    run = functools.partial(
        pl.kernel,
        out_type=jax.ShapeDtypeStruct((BATCH * SEQ, EMBED_DIM), jnp.float32),
        mesh=mesh,
        scratch_types=(
            [pltpu.VMEM((CHUNKS_PER_WORKER * SEQ,), jnp.int32),
             pltpu.VMEM((SEQ, EMBED_DIM), jnp.float32)]
            + [pltpu.VMEM((SEQ, EMBED_DIM), jnp.float32) for _ in range(NBUF)]
            + [pltpu.SemaphoreType.DMA for _ in range(2 * NBUF)]
        ),
        compiler_params=pltpu.CompilerParams(use_tc_tiling_on_sc=False),
    )(_tpe_kernel)
    out = run(inputs.reshape(-1).astype(jnp.int32), token_table, position_table)
    return out.reshape(BATCH, SEQ, EMBED_DIM)
